# baseline (speedup 1.0000x reference)
import functools

import jax
import jax.numpy as jnp
from jax import lax
from jax.experimental import pallas as pl
from jax.experimental.pallas import tpu as pltpu
from jax.experimental.pallas import tpu_sc as plsc

N = 128
K = 128
W = 256
NEG = -1e30
LN2 = 0.6931471805599453
CHUNK = 32
MB = 8
OFF = 4 * MB


def _block_update(run_max, run_sum, a_list):
    bm = a_list[0]
    for a in a_list[1:]:
        bm = jnp.maximum(bm, a)
    new_max = jnp.maximum(run_max, bm)
    acc = jnp.exp2(a_list[0] - new_max)
    for a in a_list[1:]:
        acc = acc + jnp.exp2(a - new_max)
    run_sum = run_sum * jnp.exp2(run_max - new_max) + acc
    return new_max, run_sum


def _dp_body(term_ref, rule_ref, root_ref, out_ref, d_ref):
    neg_plane = jnp.full((W, K), NEG, jnp.float32)

    def fill(l, _):
        d_ref[l] = neg_plane
        return 0

    lax.fori_loop(0, OFF + N + 1, fill, 0)
    d_ref[OFF + 1, 0:N, :] = term_ref[...]

    neg = jnp.full((CHUNK, K), NEG, jnp.float32)
    zero = jnp.zeros((CHUNK, K), jnp.float32)

    def _load_block(s, m0, c0):
        a_list = []
        for j in range(MB):
            m = m0 + j
            left = d_ref[OFF + m, c0:c0 + CHUNK, :]
            right = d_ref[OFF + s - m, pl.ds(m + c0, CHUNK), :]
            a_list.append(left + right)
        return a_list

    def _pad_row(s, w):
        if w < W:
            d_ref[OFF + s, w:W, :] = jnp.full((W - w, K), NEG, jnp.float32)

    def _pair_level_body(w):
        def run_s(s, _):
            nb = (s - 2 + 4 * MB) // (4 * MB)
            for p0 in range(0, w, 2 * CHUNK):
                cb = p0 + CHUNK

                def block(b, carry):
                    ma, sa, mb_, sb = carry
                    m0 = 1 + 4 * MB * b
                    for u in range(4):
                        mu = m0 + u * MB
                        ma, sa = _block_update(ma, sa, _load_block(s, mu, p0))
                        mb_, sb = _block_update(mb_, sb, _load_block(s, mu, cb))
                    return ma, sa, mb_, sb

                ma, sa, mb_, sb = lax.fori_loop(0, nb, block,
                                                (neg, zero, neg, zero))
                d_ref[OFF + s, p0:p0 + CHUNK, :] = (
                    ma + jnp.log2(sa) + rule_ref[...])
                d_ref[OFF + s, cb:cb + CHUNK, :] = (
                    mb_ + jnp.log2(sb) + rule_ref[...])
            _pad_row(s, w)
            return 0

        return run_s

    def _level_body(w):
        def run_s(s, _):
            nb = (s - 2 + 2 * MB) // (2 * MB)
            for c0 in range(0, w, CHUNK):
                def block(b, carry):
                    m1, s1, m2, s2 = carry
                    m0 = 1 + 2 * MB * b
                    m1, s1 = _block_update(m1, s1, _load_block(s, m0, c0))
                    m2, s2 = _block_update(m2, s2,
                                           _load_block(s, m0 + MB, c0))
                    return m1, s1, m2, s2

                m1, s1, m2, s2 = lax.fori_loop(0, nb, block,
                                               (neg, zero, neg, zero))

                run_max = jnp.maximum(m1, m2)
                run_sum = (s1 * jnp.exp2(m1 - run_max)
                           + s2 * jnp.exp2(m2 - run_max))
                vals = run_max + jnp.log2(run_sum) + rule_ref[...]
                d_ref[OFF + s, c0:c0 + CHUNK, :] = vals
            _pad_row(s, w)
            return 0

        return run_s

    b64 = max(2, N - 63)
    b32 = max(2, N - 31)
    if b64 > 2:
        lax.fori_loop(2, b64, _pair_level_body(128), 0)
    if b32 > b64:
        lax.fori_loop(b64, b32, _pair_level_body(64), 0)
    lax.fori_loop(b32, N + 1, _level_body(32), 0)

    v = d_ref[OFF + N, 0, :].reshape(1, K) + root_ref[...]
    mx = jnp.max(v)
    out = (mx + jnp.log2(jnp.sum(jnp.exp2(v - mx)))) * LN2
    out_ref[...] = out.reshape(1, 1)


def _dp(term2, rule2, root2, interpret=False):
    out = pl.pallas_call(
        _dp_body,
        in_specs=[
            pl.BlockSpec((N, K), lambda: (0, 0)),
            pl.BlockSpec((1, K), lambda: (0, 0)),
            pl.BlockSpec((1, K), lambda: (0, 0)),
        ],
        out_specs=pl.BlockSpec((1, 1), lambda: (0, 0)),
        out_shape=jax.ShapeDtypeStruct((1, 1), jnp.float32),
        scratch_shapes=[pltpu.VMEM((OFF + N + 1, W, K), jnp.float32)],
        interpret=interpret,
    )(term2, rule2, root2)
    return out[0, 0]


def _sc_gather(table, idx):
    B = idx.shape[0]
    D = table.shape[1]
    rows_per = 16
    n_workers = B // rows_per
    mesh = plsc.VectorSubcoreMesh(core_axis_name="c", subcore_axis_name="s")

    @functools.partial(
        pl.kernel,
        mesh=mesh,
        out_type=jax.ShapeDtypeStruct((B, D), jnp.float32),
        scratch_types=[
            pltpu.VMEM((rows_per,), jnp.int32),
            pltpu.VMEM((rows_per, D), jnp.float32),
            pltpu.SemaphoreType.DMA,
        ],
    )
    def k(table_hbm, idx_hbm, out_hbm, idx_v, rows_v, sem):
        wid = lax.axis_index("s") * 2 + lax.axis_index("c")

        @pl.when(wid < n_workers)
        def _():
            base = wid * rows_per
            pltpu.sync_copy(idx_hbm.at[pl.ds(base, rows_per)], idx_v)
            pltpu.async_copy(table_hbm.at[idx_v], rows_v, sem).wait()
            pltpu.sync_copy(rows_v, out_hbm.at[pl.ds(base, rows_per)])

    return k(table, idx)


def kernel(sequence, emission, rule, root):
    term = _sc_gather(emission, sequence.astype(jnp.int32))
    inv_ln2 = jnp.float32(1.0 / LN2)
    return _dp(term * inv_ln2,
               rule.reshape(1, K) * inv_ln2,
               root.reshape(1, K) * inv_ln2)


# speedup vs baseline: 1.0367x; 1.0367x over previous
import functools

import jax
import jax.numpy as jnp
from jax import lax
from jax.experimental import pallas as pl
from jax.experimental.pallas import tpu as pltpu
from jax.experimental.pallas import tpu_sc as plsc

N = 128
K = 128
W = 256
NEG = -1e30
LN2 = 0.6931471805599453
CHUNK = 32
MB = 8
OFF = 4 * MB


def _block_update(run_max, run_sum, a_list):
    bm = a_list[0]
    for a in a_list[1:]:
        bm = jnp.maximum(bm, a)
    new_max = jnp.maximum(run_max, bm)
    acc = jnp.exp2(a_list[0] - new_max)
    for a in a_list[1:]:
        acc = acc + jnp.exp2(a - new_max)
    run_sum = run_sum * jnp.exp2(run_max - new_max) + acc
    return new_max, run_sum


def _dp_body(term_ref, rule_ref, root_ref, out_ref, d_ref):
    neg_plane = jnp.full((W, K), NEG, jnp.float32)

    def fill(l, _):
        d_ref[l] = neg_plane
        return 0

    lax.fori_loop(0, OFF + N + 1, fill, 0)
    d_ref[OFF + 1, 0:N, :] = term_ref[...]

    neg = jnp.full((CHUNK, K), NEG, jnp.float32)
    zero = jnp.zeros((CHUNK, K), jnp.float32)

    def _load_block(s, m0, c0):
        a_list = []
        for j in range(MB):
            m = m0 + j
            left = d_ref[OFF + m, c0:c0 + CHUNK, :]
            right = d_ref[OFF + s - m, pl.ds(m + c0, CHUNK), :]
            a_list.append(left + right)
        return a_list

    def _pad_row(s, w):
        if w < W:
            d_ref[OFF + s, w:W, :] = jnp.full((W - w, K), NEG, jnp.float32)

    def _pair_level_body(w):
        def run_s(s, _):
            nb = (s - 2 + 2 * MB) // (2 * MB)
            for p0 in range(0, w, 2 * CHUNK):
                cb = p0 + CHUNK

                def block(b, carry):
                    ma, sa, mb_, sb = carry
                    m0 = 1 + 2 * MB * b
                    ma, sa = _block_update(ma, sa, _load_block(s, m0, p0))
                    mb_, sb = _block_update(mb_, sb, _load_block(s, m0, cb))
                    ma, sa = _block_update(ma, sa, _load_block(s, m0 + MB, p0))
                    mb_, sb = _block_update(mb_, sb, _load_block(s, m0 + MB, cb))
                    return ma, sa, mb_, sb

                ma, sa, mb_, sb = lax.fori_loop(0, nb, block,
                                                (neg, zero, neg, zero))
                d_ref[OFF + s, p0:p0 + CHUNK, :] = (
                    ma + jnp.log2(sa) + rule_ref[...])
                d_ref[OFF + s, cb:cb + CHUNK, :] = (
                    mb_ + jnp.log2(sb) + rule_ref[...])
            _pad_row(s, w)
            return 0

        return run_s

    def _level_body(w):
        def run_s(s, _):
            nb = (s - 2 + 4 * MB) // (4 * MB)
            for c0 in range(0, w, CHUNK):
                def block(b, carry):
                    m1, s1, m2, s2 = carry
                    m0 = 1 + 4 * MB * b
                    m1, s1 = _block_update(m1, s1, _load_block(s, m0, c0))
                    m2, s2 = _block_update(m2, s2,
                                           _load_block(s, m0 + MB, c0))
                    m1, s1 = _block_update(m1, s1,
                                           _load_block(s, m0 + 2 * MB, c0))
                    m2, s2 = _block_update(m2, s2,
                                           _load_block(s, m0 + 3 * MB, c0))
                    return m1, s1, m2, s2

                m1, s1, m2, s2 = lax.fori_loop(0, nb, block,
                                               (neg, zero, neg, zero))

                run_max = jnp.maximum(m1, m2)
                run_sum = (s1 * jnp.exp2(m1 - run_max)
                           + s2 * jnp.exp2(m2 - run_max))
                vals = run_max + jnp.log2(run_sum) + rule_ref[...]
                d_ref[OFF + s, c0:c0 + CHUNK, :] = vals
            _pad_row(s, w)
            return 0

        return run_s

    b64 = max(2, N - 63)
    b32 = max(2, N - 31)
    if b64 > 2:
        lax.fori_loop(2, b64, _pair_level_body(128), 0)
    if b32 > b64:
        lax.fori_loop(b64, b32, _pair_level_body(64), 0)
    lax.fori_loop(b32, N + 1, _level_body(32), 0)

    v = d_ref[OFF + N, 0, :].reshape(1, K) + root_ref[...]
    mx = jnp.max(v)
    out = (mx + jnp.log2(jnp.sum(jnp.exp2(v - mx)))) * LN2
    out_ref[...] = out.reshape(1, 1)


def _dp(term2, rule2, root2, interpret=False):
    out = pl.pallas_call(
        _dp_body,
        in_specs=[
            pl.BlockSpec((N, K), lambda: (0, 0)),
            pl.BlockSpec((1, K), lambda: (0, 0)),
            pl.BlockSpec((1, K), lambda: (0, 0)),
        ],
        out_specs=pl.BlockSpec((1, 1), lambda: (0, 0)),
        out_shape=jax.ShapeDtypeStruct((1, 1), jnp.float32),
        scratch_shapes=[pltpu.VMEM((OFF + N + 1, W, K), jnp.float32)],
        interpret=interpret,
    )(term2, rule2, root2)
    return out[0, 0]


def _sc_gather(table, idx):
    B = idx.shape[0]
    D = table.shape[1]
    rows_per = 16
    n_workers = B // rows_per
    mesh = plsc.VectorSubcoreMesh(core_axis_name="c", subcore_axis_name="s")

    @functools.partial(
        pl.kernel,
        mesh=mesh,
        out_type=jax.ShapeDtypeStruct((B, D), jnp.float32),
        scratch_types=[
            pltpu.VMEM((rows_per,), jnp.int32),
            pltpu.VMEM((rows_per, D), jnp.float32),
            pltpu.SemaphoreType.DMA,
        ],
    )
    def k(table_hbm, idx_hbm, out_hbm, idx_v, rows_v, sem):
        wid = lax.axis_index("s") * 2 + lax.axis_index("c")

        @pl.when(wid < n_workers)
        def _():
            base = wid * rows_per
            pltpu.sync_copy(idx_hbm.at[pl.ds(base, rows_per)], idx_v)
            pltpu.async_copy(table_hbm.at[idx_v], rows_v, sem).wait()
            pltpu.sync_copy(rows_v, out_hbm.at[pl.ds(base, rows_per)])

    return k(table, idx)


def kernel(sequence, emission, rule, root):
    term = _sc_gather(emission, sequence.astype(jnp.int32))
    inv_ln2 = jnp.float32(1.0 / LN2)
    return _dp(term * inv_ln2,
               rule.reshape(1, K) * inv_ln2,
               root.reshape(1, K) * inv_ln2)


# speedup vs baseline: 1.0664x; 1.0286x over previous
import functools

import jax
import jax.numpy as jnp
from jax import lax
from jax.experimental import pallas as pl
from jax.experimental.pallas import tpu as pltpu
from jax.experimental.pallas import tpu_sc as plsc

N = 128
K = 128
W = 256
NEG = -1e30
LN2 = 0.6931471805599453
CHUNK = 32
MB = 8
OFF = 4 * MB


def _block_update(run_max, run_sum, a_list):
    bm = a_list[0]
    for a in a_list[1:]:
        bm = jnp.maximum(bm, a)
    new_max = jnp.maximum(run_max, bm)
    acc = jnp.exp2(a_list[0] - new_max)
    for a in a_list[1:]:
        acc = acc + jnp.exp2(a - new_max)
    run_sum = run_sum * jnp.exp2(run_max - new_max) + acc
    return new_max, run_sum


def _dp_body(term_ref, rule_ref, root_ref, out_ref, d_ref):
    neg_plane = jnp.full((W, K), NEG, jnp.float32)

    def fill(l, _):
        d_ref[l] = neg_plane
        return 0

    lax.fori_loop(0, OFF + N + 1, fill, 0)
    d_ref[OFF + 1, 0:N, :] = term_ref[...]

    neg = jnp.full((CHUNK, K), NEG, jnp.float32)
    zero = jnp.zeros((CHUNK, K), jnp.float32)

    def _load_block(s, m0, c0, ch=CHUNK):
        a_list = []
        for j in range(MB):
            m = m0 + j
            left = d_ref[OFF + m, c0:c0 + ch, :]
            right = d_ref[OFF + s - m, pl.ds(m + c0, ch), :]
            a_list.append(left + right)
        return a_list

    def _pad_row(s, w):
        if w < W:
            d_ref[OFF + s, w:W, :] = jnp.full((W - w, K), NEG, jnp.float32)

    def _pair_level_body(w):
        def run_s(s, _):
            nb = (s - 2 + 2 * MB) // (2 * MB)
            for p0 in range(0, w, 2 * CHUNK):
                cb = p0 + CHUNK

                def block(b, carry):
                    ma, sa, mb_, sb = carry
                    m0 = 1 + 2 * MB * b
                    ma, sa = _block_update(ma, sa, _load_block(s, m0, p0))
                    mb_, sb = _block_update(mb_, sb, _load_block(s, m0, cb))
                    ma, sa = _block_update(ma, sa, _load_block(s, m0 + MB, p0))
                    mb_, sb = _block_update(mb_, sb, _load_block(s, m0 + MB, cb))
                    return ma, sa, mb_, sb

                ma, sa, mb_, sb = lax.fori_loop(0, nb, block,
                                                (neg, zero, neg, zero))
                d_ref[OFF + s, p0:p0 + CHUNK, :] = (
                    ma + jnp.log2(sa) + rule_ref[...])
                d_ref[OFF + s, cb:cb + CHUNK, :] = (
                    mb_ + jnp.log2(sb) + rule_ref[...])
            _pad_row(s, w)
            return 0

        return run_s

    def _level_body(w, ch=CHUNK):
        negc = jnp.full((ch, K), NEG, jnp.float32)
        zeroc = jnp.zeros((ch, K), jnp.float32)

        def run_s(s, _):
            nb = (s - 2 + 4 * MB) // (4 * MB)
            for c0 in range(0, w, ch):
                def block(b, carry):
                    m1, s1, m2, s2 = carry
                    m0 = 1 + 4 * MB * b
                    m1, s1 = _block_update(m1, s1,
                                           _load_block(s, m0, c0, ch))
                    m2, s2 = _block_update(m2, s2,
                                           _load_block(s, m0 + MB, c0, ch))
                    m1, s1 = _block_update(m1, s1,
                                           _load_block(s, m0 + 2 * MB, c0,
                                                       ch))
                    m2, s2 = _block_update(m2, s2,
                                           _load_block(s, m0 + 3 * MB, c0,
                                                       ch))
                    return m1, s1, m2, s2

                m1, s1, m2, s2 = lax.fori_loop(0, nb, block,
                                               (negc, zeroc, negc, zeroc))

                run_max = jnp.maximum(m1, m2)
                run_sum = (s1 * jnp.exp2(m1 - run_max)
                           + s2 * jnp.exp2(m2 - run_max))
                vals = run_max + jnp.log2(run_sum) + rule_ref[...]
                d_ref[OFF + s, c0:c0 + ch, :] = vals
            _pad_row(s, w)
            return 0

        return run_s

    b64 = max(2, N - 63)
    b32 = max(2, N - 31)
    b16 = max(2, N - 15)
    if b64 > 2:
        lax.fori_loop(2, b64, _pair_level_body(128), 0)
    if b32 > b64:
        lax.fori_loop(b64, b32, _pair_level_body(64), 0)
    if b16 > b32:
        lax.fori_loop(b32, b16, _level_body(32), 0)
    lax.fori_loop(b16, N + 1, _level_body(16, 16), 0)

    v = d_ref[OFF + N, 0, :].reshape(1, K) + root_ref[...]
    mx = jnp.max(v)
    out = (mx + jnp.log2(jnp.sum(jnp.exp2(v - mx)))) * LN2
    out_ref[...] = out.reshape(1, 1)


def _dp(term2, rule2, root2, interpret=False):
    out = pl.pallas_call(
        _dp_body,
        in_specs=[
            pl.BlockSpec((N, K), lambda: (0, 0)),
            pl.BlockSpec((1, K), lambda: (0, 0)),
            pl.BlockSpec((1, K), lambda: (0, 0)),
        ],
        out_specs=pl.BlockSpec((1, 1), lambda: (0, 0)),
        out_shape=jax.ShapeDtypeStruct((1, 1), jnp.float32),
        scratch_shapes=[pltpu.VMEM((OFF + N + 1, W, K), jnp.float32)],
        interpret=interpret,
    )(term2, rule2, root2)
    return out[0, 0]


def _sc_gather(table, idx):
    B = idx.shape[0]
    D = table.shape[1]
    rows_per = 16
    n_workers = B // rows_per
    mesh = plsc.VectorSubcoreMesh(core_axis_name="c", subcore_axis_name="s")

    @functools.partial(
        pl.kernel,
        mesh=mesh,
        out_type=jax.ShapeDtypeStruct((B, D), jnp.float32),
        scratch_types=[
            pltpu.VMEM((rows_per,), jnp.int32),
            pltpu.VMEM((rows_per, D), jnp.float32),
            pltpu.SemaphoreType.DMA,
        ],
    )
    def k(table_hbm, idx_hbm, out_hbm, idx_v, rows_v, sem):
        wid = lax.axis_index("s") * 2 + lax.axis_index("c")

        @pl.when(wid < n_workers)
        def _():
            base = wid * rows_per
            pltpu.sync_copy(idx_hbm.at[pl.ds(base, rows_per)], idx_v)
            pltpu.async_copy(table_hbm.at[idx_v], rows_v, sem).wait()
            pltpu.sync_copy(rows_v, out_hbm.at[pl.ds(base, rows_per)])

    return k(table, idx)


def kernel(sequence, emission, rule, root):
    term = _sc_gather(emission, sequence.astype(jnp.int32))
    inv_ln2 = jnp.float32(1.0 / LN2)
    return _dp(term * inv_ln2,
               rule.reshape(1, K) * inv_ln2,
               root.reshape(1, K) * inv_ln2)


# speedup vs baseline: 1.0852x; 1.0176x over previous
import functools

import jax
import jax.numpy as jnp
from jax import lax
from jax.experimental import pallas as pl
from jax.experimental.pallas import tpu as pltpu
from jax.experimental.pallas import tpu_sc as plsc

N = 128
K = 128
W = 256
NEG = -1e30
LN2 = 0.6931471805599453
CHUNK = 32
MB = 8
OFF = 4 * MB


def _block_update(run_max, run_sum, a_list):
    bm = a_list[0]
    for a in a_list[1:]:
        bm = jnp.maximum(bm, a)
    new_max = jnp.maximum(run_max, bm)
    acc = jnp.exp2(a_list[0] - new_max)
    for a in a_list[1:]:
        acc = acc + jnp.exp2(a - new_max)
    run_sum = run_sum * jnp.exp2(run_max - new_max) + acc
    return new_max, run_sum


def _dp_body(term_ref, rule_ref, root_ref, out_ref, d_ref):
    neg_plane = jnp.full((W, K), NEG, jnp.float32)

    def fill(l, _):
        d_ref[l] = neg_plane
        return 0

    inv_ln2 = jnp.float32(1.0 / LN2)
    lax.fori_loop(0, OFF + N + 1, fill, 0)
    d_ref[OFF + 1, 0:N, :] = term_ref[...] * inv_ln2
    rule2 = rule_ref[...] * inv_ln2

    neg = jnp.full((CHUNK, K), NEG, jnp.float32)
    zero = jnp.zeros((CHUNK, K), jnp.float32)

    def _load_block(s, m0, c0, ch=CHUNK):
        a_list = []
        for j in range(MB):
            m = m0 + j
            left = d_ref[OFF + m, c0:c0 + ch, :]
            right = d_ref[OFF + s - m, pl.ds(m + c0, ch), :]
            a_list.append(left + right)
        return a_list

    def _pad_row(s, w):
        if w < W:
            d_ref[OFF + s, w:W, :] = jnp.full((W - w, K), NEG, jnp.float32)

    def _pair_level_body(w):
        def run_s(s, _):
            nb = (s - 2 + 2 * MB) // (2 * MB)
            for p0 in range(0, w, 2 * CHUNK):
                cb = p0 + CHUNK

                def block(b, carry):
                    ma, sa, mb_, sb = carry
                    m0 = 1 + 2 * MB * b
                    ma, sa = _block_update(ma, sa, _load_block(s, m0, p0))
                    mb_, sb = _block_update(mb_, sb, _load_block(s, m0, cb))
                    ma, sa = _block_update(ma, sa, _load_block(s, m0 + MB, p0))
                    mb_, sb = _block_update(mb_, sb, _load_block(s, m0 + MB, cb))
                    return ma, sa, mb_, sb

                ma, sa, mb_, sb = lax.fori_loop(0, nb, block,
                                                (neg, zero, neg, zero))
                d_ref[OFF + s, p0:p0 + CHUNK, :] = (
                    ma + jnp.log2(sa) + rule2)
                d_ref[OFF + s, cb:cb + CHUNK, :] = (
                    mb_ + jnp.log2(sb) + rule2)
            _pad_row(s, w)
            return 0

        return run_s

    def _level_body(w, ch=CHUNK):
        negc = jnp.full((ch, K), NEG, jnp.float32)
        zeroc = jnp.zeros((ch, K), jnp.float32)

        def run_s(s, _):
            nb = (s - 2 + 4 * MB) // (4 * MB)
            for c0 in range(0, w, ch):
                def block(b, carry):
                    m1, s1, m2, s2 = carry
                    m0 = 1 + 4 * MB * b
                    m1, s1 = _block_update(m1, s1,
                                           _load_block(s, m0, c0, ch))
                    m2, s2 = _block_update(m2, s2,
                                           _load_block(s, m0 + MB, c0, ch))
                    m1, s1 = _block_update(m1, s1,
                                           _load_block(s, m0 + 2 * MB, c0,
                                                       ch))
                    m2, s2 = _block_update(m2, s2,
                                           _load_block(s, m0 + 3 * MB, c0,
                                                       ch))
                    return m1, s1, m2, s2

                m1, s1, m2, s2 = lax.fori_loop(0, nb, block,
                                               (negc, zeroc, negc, zeroc))

                run_max = jnp.maximum(m1, m2)
                run_sum = (s1 * jnp.exp2(m1 - run_max)
                           + s2 * jnp.exp2(m2 - run_max))
                vals = run_max + jnp.log2(run_sum) + rule2
                d_ref[OFF + s, c0:c0 + ch, :] = vals
            _pad_row(s, w)
            return 0

        return run_s

    b64 = max(2, N - 63)
    b32 = max(2, N - 31)
    b16 = max(2, N - 15)
    if b64 > 2:
        lax.fori_loop(2, b64, _pair_level_body(128), 0)
    if b32 > b64:
        lax.fori_loop(b64, b32, _pair_level_body(64), 0)
    if b16 > b32:
        lax.fori_loop(b32, b16, _level_body(32), 0)
    lax.fori_loop(b16, N + 1, _level_body(16, 16), 0)

    v = d_ref[OFF + N, 0, :].reshape(1, K) + root_ref[...] * inv_ln2
    mx = jnp.max(v)
    out = (mx + jnp.log2(jnp.sum(jnp.exp2(v - mx)))) * LN2
    out_ref[...] = out.reshape(1, 1)


def _dp(term2, rule2, root2, interpret=False):
    out = pl.pallas_call(
        _dp_body,
        in_specs=[
            pl.BlockSpec((N, K), lambda: (0, 0)),
            pl.BlockSpec((1, K), lambda: (0, 0)),
            pl.BlockSpec((1, K), lambda: (0, 0)),
        ],
        out_specs=pl.BlockSpec((1, 1), lambda: (0, 0)),
        out_shape=jax.ShapeDtypeStruct((1, 1), jnp.float32),
        scratch_shapes=[pltpu.VMEM((OFF + N + 1, W, K), jnp.float32)],
        interpret=interpret,
    )(term2, rule2, root2)
    return out[0, 0]


def _sc_gather(table, idx):
    B = idx.shape[0]
    D = table.shape[1]
    rows_per = 16
    n_workers = B // rows_per
    mesh = plsc.VectorSubcoreMesh(core_axis_name="c", subcore_axis_name="s")

    @functools.partial(
        pl.kernel,
        mesh=mesh,
        out_type=jax.ShapeDtypeStruct((B, D), jnp.float32),
        scratch_types=[
            pltpu.VMEM((rows_per,), jnp.int32),
            pltpu.VMEM((rows_per, D), jnp.float32),
            pltpu.SemaphoreType.DMA,
        ],
    )
    def k(table_hbm, idx_hbm, out_hbm, idx_v, rows_v, sem):
        wid = lax.axis_index("s") * 2 + lax.axis_index("c")

        @pl.when(wid < n_workers)
        def _():
            base = wid * rows_per
            pltpu.sync_copy(idx_hbm.at[pl.ds(base, rows_per)], idx_v)
            pltpu.async_copy(table_hbm.at[idx_v], rows_v, sem).wait()
            pltpu.sync_copy(rows_v, out_hbm.at[pl.ds(base, rows_per)])

    return k(table, idx)


def kernel(sequence, emission, rule, root):
    term = _sc_gather(emission, sequence.astype(jnp.int32))
    return _dp(term, rule.reshape(1, K), root.reshape(1, K))
